# baseline (speedup 1.0000x reference)
import functools

import jax
import jax.numpy as jnp
from jax import lax
from jax.experimental import pallas as pl
from jax.experimental.pallas import tpu as pltpu
from jax.experimental.pallas import tpu_sc as plsc

VOCAB = 1000000
EMB = 64
B = 4096
T = 200
BTOT = B * T

_info = plsc.get_sparse_core_info()
NC, NS = _info.num_cores, _info.num_subcores
NW = NC * NS

K1_UNITS = VOCAB // 128
TAIL_V0 = K1_UNITS * 128
TAIL_W = VOCAB - TAIL_V0
TAIL_WID = K1_UNITS % NW
K1_KMAX = -(-K1_UNITS // NW)

BBLK = B // NW


def _k1():
    mesh = plsc.VectorSubcoreMesh(core_axis_name="c", subcore_axis_name="s")

    @functools.partial(
        pl.kernel,
        out_type=jax.ShapeDtypeStruct((VOCAB * EMB,), jnp.float32),
        mesh=mesh,
        scratch_types=[
            pltpu.VMEM((EMB, 128), jnp.float32),
            pltpu.VMEM((EMB, 128), jnp.float32),
            pltpu.VMEM((128 * EMB,), jnp.float32),
            pltpu.VMEM((128 * EMB,), jnp.float32),
            pltpu.SemaphoreType.DMA,
            pltpu.SemaphoreType.DMA,
            pltpu.SemaphoreType.DMA,
            pltpu.SemaphoreType.DMA,
        ],
        compiler_params=pltpu.CompilerParams(
            use_tc_tiling_on_sc=True, needs_layout_passes=False
        ),
    )
    def body(tabt, tail, flat, slab_a, slab_b, stag_a, stag_b, ia, ib, oa, ob):
        c = lax.axis_index("c")
        s = lax.axis_index("s")
        wid = s * NC + c
        lane64 = lax.iota(jnp.int32, 16) * EMB

        slabs = (slab_a, slab_b)
        stags = (stag_a, stag_b)
        isems = (ia, ib)
        osems = (oa, ob)

        def v0_of(k):
            return (wid + NW * k) * 128

        def in_range(k):
            return (k >= 0) & (wid + NW * k < K1_UNITS)

        def issue_slab(k, buf):
            @pl.when(in_range(k))
            def _():
                pltpu.async_copy(
                    tabt.at[:, pl.ds(v0_of(k), 128)], slabs[buf], isems[buf]
                )

        def drain_slab(k, buf):
            @pl.when(in_range(k))
            def _():
                pltpu.make_async_copy(
                    tabt.at[:, pl.ds(v0_of(k), 128)], slabs[buf], isems[buf]
                ).wait()

        def issue_stag(k, buf):
            @pl.when(in_range(k))
            def _():
                pltpu.async_copy(
                    stags[buf], flat.at[pl.ds(v0_of(k) * EMB, 128 * EMB)],
                    osems[buf],
                )

        def drain_stag(k, buf):
            @pl.when(in_range(k))
            def _():
                pltpu.make_async_copy(
                    stags[buf], flat.at[pl.ds(v0_of(k) * EMB, 128 * EMB)],
                    osems[buf],
                ).wait()

        def transpose(buf):
            slab, stag = slabs[buf], stags[buf]
            for e in range(EMB):
                for c0 in range(0, 128, 16):
                    vals = slab[e, pl.ds(c0, 16)]
                    plsc.store_scatter(stag, [lane64 + (c0 * EMB + e)], vals)

        issue_slab(0, 0)

        def step(k2, _):
            for bown in range(2):
                k = 2 * k2 + bown
                drain_slab(k, bown)
                issue_slab(k + 1, 1 - bown)
                drain_stag(k - 2, bown)

                @pl.when(in_range(k))
                def _():
                    transpose(bown)
                issue_stag(k, bown)
            return 0

        lax.fori_loop(0, (K1_KMAX + 1) // 2, step, 0, unroll=False)
        kend = 2 * ((K1_KMAX + 1) // 2)
        drain_stag(kend - 2, 0)
        drain_stag(kend - 1, 1)

        @pl.when(wid == TAIL_WID)
        def _():
            pltpu.sync_copy(tail, stag_a.at[pl.ds(0, TAIL_W * EMB)])
            pltpu.sync_copy(
                stag_a.at[pl.ds(0, TAIL_W * EMB)],
                flat.at[pl.ds(TAIL_V0 * EMB, TAIL_W * EMB)],
            )

    return body


def _k2():
    mesh = plsc.VectorSubcoreMesh(core_axis_name="c", subcore_axis_name="s")

    @functools.partial(
        pl.kernel,
        out_type=jax.ShapeDtypeStruct((T, 8, B // 128, 8, 128), jnp.float32),
        mesh=mesh,
        scratch_types=[
            pltpu.VMEM((T, BBLK), jnp.int32),
            pltpu.VMEM((BBLK, EMB), jnp.float32),
            pltpu.VMEM((BBLK, EMB), jnp.float32),
            pltpu.VMEM((8, 1, 8, 128), jnp.float32),
            pltpu.VMEM((8, 1, 8, 128), jnp.float32),
            pltpu.SemaphoreType.DMA,
            pltpu.SemaphoreType.DMA,
            pltpu.SemaphoreType.DMA,
            pltpu.SemaphoreType.DMA,
        ],
        compiler_params=pltpu.CompilerParams(
            use_tc_tiling_on_sc=False, needs_layout_passes=False
        ),
    )
    def body(xflat, tab, out5, idxall, gb_a, gb_b, st_a, st_b, ga, gbs, oa, ob):
        c = lax.axis_index("c")
        s = lax.axis_index("s")
        wid = s * NC + c
        iota = lax.iota(jnp.int32, 16)

        gbufs = (gb_a, gb_b)
        stags = (st_a, st_b)
        gsems = (ga, gbs)
        osems = (oa, ob)

        def ld(t, _):
            pltpu.sync_copy(
                xflat.at[pl.ds(t * B + wid * BBLK, BBLK)], idxall.at[t]
            )
            return 0
        lax.fori_loop(0, T, ld, 0, unroll=False)

        bcols = [c0 + iota for c0 in range(0, BBLK, 16)]

        def guard(t):
            return (t >= 0) & (t < T)

        def tsafe(t):
            return jnp.clip(t, 0, T - 1)

        def issue_gather(t, buf):
            @pl.when(guard(t))
            def _():
                pltpu.async_copy(
                    tab.at[idxall.at[tsafe(t)]], gbufs[buf], gsems[buf]
                )

        def drain_gather(t, buf):
            @pl.when(guard(t))
            def _():
                pltpu.make_async_copy(
                    tab.at[idxall.at[tsafe(t)]], gbufs[buf], gsems[buf]
                ).wait()

        def issue_out(t, buf):
            @pl.when(guard(t))
            def _():
                pltpu.async_copy(
                    stags[buf], out5.at[tsafe(t), :, pl.ds(wid, 1), :, :],
                    osems[buf],
                )

        def drain_out(t, buf):
            @pl.when(guard(t))
            def _():
                pltpu.make_async_copy(
                    stags[buf], out5.at[tsafe(t), :, pl.ds(wid, 1), :, :],
                    osems[buf],
                ).wait()

        def transpose(buf):
            gbuf = gbufs[buf]
            stag = stags[buf]
            for e in range(EMB):
                esplat = iota * 0 + e
                for j, c0 in enumerate(range(0, BBLK, 16)):
                    vals = plsc.load_gather(gbuf, [bcols[j], esplat])
                    stag[e // 8, 0, e % 8, pl.ds(c0, 16)] = vals

        issue_gather(0, 0)

        def step(k2i, _):
            for bown in range(2):
                t = 2 * k2i + bown
                drain_gather(t, bown)
                issue_gather(t + 1, 1 - bown)
                drain_out(t - 2, bown)

                @pl.when(t < T)
                def _():
                    transpose(bown)
                issue_out(t, bown)
            return 0

        lax.fori_loop(0, T // 2, step, 0, unroll=False)
        drain_out(T - 2, 0)
        drain_out(T - 1, 1)

    return body


_transpose_table = _k1()
_gather_blocks = _k2()


@jax.jit
def kernel(x, table):
    tabt = jnp.transpose(table)
    tail_lin = table[TAIL_V0:, :].reshape(TAIL_W * EMB)
    flat = _transpose_table(tabt, tail_lin)
    tab_lin = flat.reshape(VOCAB, EMB)
    xflat = jnp.transpose(x).reshape(BTOT)
    out5 = _gather_blocks(xflat, tab_lin)
    return jnp.transpose(out5, (2, 4, 0, 1, 3)).reshape(B, T, EMB)


# speedup vs baseline: 1.4431x; 1.4431x over previous
import functools

import jax
import jax.numpy as jnp
from jax import lax
from jax.experimental import pallas as pl
from jax.experimental.pallas import tpu as pltpu
from jax.experimental.pallas import tpu_sc as plsc

VOCAB = 1000000
EMB = 64
B = 4096
T = 200
BTOT = B * T

_info = plsc.get_sparse_core_info()
NC, NS = _info.num_cores, _info.num_subcores
NW = NC * NS

K1_UNITS = VOCAB // 128
TAIL_V0 = K1_UNITS * 128
TAIL_W = VOCAB - TAIL_V0
TAIL_WID = K1_UNITS % NW
K1_KMAX = -(-K1_UNITS // NW)

BBLK = B // NW


def _k1():
    mesh = plsc.VectorSubcoreMesh(core_axis_name="c", subcore_axis_name="s")

    @functools.partial(
        pl.kernel,
        out_type=jax.ShapeDtypeStruct((VOCAB * EMB,), jnp.float32),
        mesh=mesh,
        scratch_types=(
            [pltpu.VMEM((8, 128), jnp.float32) for _ in range(16)]
            + [
                pltpu.VMEM((8256,), jnp.float32),
                pltpu.VMEM((8256,), jnp.float32),
                pltpu.SemaphoreType.DMA,
                pltpu.SemaphoreType.DMA,
                pltpu.SemaphoreType.DMA,
                pltpu.SemaphoreType.DMA,
            ]
        ),
        compiler_params=pltpu.CompilerParams(
            use_tc_tiling_on_sc=True, needs_layout_passes=False
        ),
    )
    def body(tabt, tail, flat, *refs):
        slabs = (refs[0:8], refs[8:16])
        stag_a, stag_b = refs[16], refs[17]
        ia, ib, oa, ob = refs[18], refs[19], refs[20], refs[21]
        c = lax.axis_index("c")
        s = lax.axis_index("s")
        wid = s * NC + c
        lane64 = lax.iota(jnp.int32, 16) * EMB

        stags = (stag_a, stag_b)
        isems = (ia, ib)
        osems = (oa, ob)

        def v0_of(k):
            return (wid + NW * k) * 128

        def in_range(k):
            return (k >= 0) & (wid + NW * k < K1_UNITS)

        def issue_slab(k, buf):
            @pl.when(in_range(k))
            def _():
                for tr in range(8):
                    pltpu.async_copy(
                        tabt.at[pl.ds(8 * tr, 8), pl.ds(v0_of(k), 128)],
                        slabs[buf][tr], isems[buf],
                    )

        def drain_slab(k, buf):
            @pl.when(in_range(k))
            def _():
                for tr in range(8):
                    pltpu.make_async_copy(
                        tabt.at[pl.ds(8 * tr, 8), pl.ds(v0_of(k), 128)],
                        slabs[buf][tr], isems[buf],
                    ).wait()

        def issue_stag(k, buf):
            @pl.when(in_range(k))
            def _():
                pltpu.async_copy(
                    stags[buf].at[pl.ds(0, 128 * EMB)],
                    flat.at[pl.ds(v0_of(k) * EMB, 128 * EMB)],
                    osems[buf],
                )

        def drain_stag(k, buf):
            @pl.when(in_range(k))
            def _():
                pltpu.make_async_copy(
                    stags[buf].at[pl.ds(0, 128 * EMB)],
                    flat.at[pl.ds(v0_of(k) * EMB, 128 * EMB)],
                    osems[buf],
                ).wait()

        def transpose(buf):
            stag = stags[buf]
            for tr in range(8):
                tile = slabs[buf][tr]

                @plsc.parallel_loop(0, 128, step=16, unroll=8)
                def _(c0):
                    c0m = pl.multiple_of(c0, 16)
                    base = pl.multiple_of(8 * tr + c0m * EMB, 8)
                    for e in range(8):
                        vals = tile[e, pl.ds(c0m, 16)]
                        plsc.store_scatter(
                            stag.at[pl.ds(base, 1024)], [lane64 + e], vals
                        )

        issue_slab(0, 0)

        def step(k2, _):
            for bown in range(2):
                k = 2 * k2 + bown
                drain_slab(k, bown)
                issue_slab(k + 1, 1 - bown)
                drain_stag(k - 2, bown)

                @pl.when(in_range(k))
                def _():
                    transpose(bown)
                issue_stag(k, bown)
            return 0

        lax.fori_loop(0, (K1_KMAX + 1) // 2, step, 0, unroll=False)
        kend = 2 * ((K1_KMAX + 1) // 2)
        drain_stag(kend - 2, 0)
        drain_stag(kend - 1, 1)

        @pl.when(wid == TAIL_WID)
        def _():
            pltpu.sync_copy(tail, stag_a.at[pl.ds(0, TAIL_W * EMB)])
            pltpu.sync_copy(
                stag_a.at[pl.ds(0, TAIL_W * EMB)],
                flat.at[pl.ds(TAIL_V0 * EMB, TAIL_W * EMB)],
            )

    return body


def _k2():
    mesh = plsc.VectorSubcoreMesh(core_axis_name="c", subcore_axis_name="s")

    @functools.partial(
        pl.kernel,
        out_type=jax.ShapeDtypeStruct((T * EMB * B,), jnp.float32),
        mesh=mesh,
        scratch_types=[
            pltpu.VMEM((T, BBLK), jnp.int32),
            pltpu.VMEM((BBLK, EMB), jnp.float32),
            pltpu.VMEM((BBLK, EMB), jnp.float32),
            pltpu.VMEM((8 * 8 * 128,), jnp.float32),
            pltpu.VMEM((8 * 8 * 128,), jnp.float32),
            pltpu.SemaphoreType.DMA,
            pltpu.SemaphoreType.DMA,
            pltpu.SemaphoreType.DMA,
            pltpu.SemaphoreType.DMA,
        ],
        compiler_params=pltpu.CompilerParams(
            use_tc_tiling_on_sc=False, needs_layout_passes=False
        ),
    )
    def body(xflat, tab, outf, idxall, gb_a, gb_b, st_a, st_b, ga, gbs, oa, ob):
        c = lax.axis_index("c")
        s = lax.axis_index("s")
        wid = s * NC + c
        iota = lax.iota(jnp.int32, 16)

        gbufs = (gb_a, gb_b)
        stags = (st_a, st_b)
        gsems = (ga, gbs)
        osems = (oa, ob)

        def ld(t, _):
            pltpu.sync_copy(
                xflat.at[pl.ds(t * B + wid * BBLK, BBLK)], idxall.at[t]
            )
            return 0
        lax.fori_loop(0, T, ld, 0, unroll=False)

        consts = []
        for e0 in range(0, EMB, 16):
            e = e0 + iota
            consts.append(
                jax.lax.shift_right_logical(e, 3) * 1024
                + jnp.bitwise_and(e, 7) * 128
            )

        def guard(t):
            return (t >= 0) & (t < T)

        def tsafe(t):
            return jnp.clip(t, 0, T - 1)

        def issue_gather(t, buf):
            @pl.when(guard(t))
            def _():
                pltpu.async_copy(
                    tab.at[idxall.at[tsafe(t)]], gbufs[buf], gsems[buf]
                )

        def drain_gather(t, buf):
            @pl.when(guard(t))
            def _():
                pltpu.make_async_copy(
                    tab.at[idxall.at[tsafe(t)]], gbufs[buf], gsems[buf]
                ).wait()

        def obase(t):
            return tsafe(t) * (EMB * B) + wid * 1024

        def issue_out(t, buf):
            @pl.when(guard(t))
            def _():
                for et in range(8):
                    pltpu.async_copy(
                        stags[buf].at[pl.ds(et * 1024, 1024)],
                        outf.at[pl.ds(obase(t) + et * 32768, 1024)],
                        osems[buf],
                    )

        def drain_out(t, buf):
            @pl.when(guard(t))
            def _():
                for et in range(8):
                    pltpu.make_async_copy(
                        stags[buf].at[pl.ds(et * 1024, 1024)],
                        outf.at[pl.ds(obase(t) + et * 32768, 1024)],
                        osems[buf],
                    ).wait()

        def transpose(buf):
            gbuf = gbufs[buf]
            stag = stags[buf]

            @plsc.parallel_loop(0, BBLK, step=1, unroll=8)
            def _(b1):
                for j, e0 in enumerate(range(0, EMB, 16)):
                    vals = gbuf[b1, pl.ds(e0, 16)]
                    plsc.store_scatter(stag, [consts[j] + b1], vals)

        issue_gather(0, 0)

        def step(k2i, _):
            for bown in range(2):
                t = 2 * k2i + bown
                drain_gather(t, bown)
                issue_gather(t + 1, 1 - bown)
                drain_out(t - 2, bown)

                @pl.when(t < T)
                def _():
                    transpose(bown)
                issue_out(t, bown)
            return 0

        lax.fori_loop(0, T // 2, step, 0, unroll=False)
        drain_out(T - 2, 0)
        drain_out(T - 1, 1)

    return body


_transpose_table = _k1()
_gather_blocks = _k2()


@jax.jit
def kernel(x, table):
    tabt = jnp.transpose(table)
    tail_lin = table[TAIL_V0:, :].reshape(TAIL_W * EMB)
    flat = _transpose_table(tabt, tail_lin)
    tab_lin = flat.reshape(VOCAB, EMB)
    xflat = jnp.transpose(x).reshape(BTOT)
    outf = _gather_blocks(xflat, tab_lin)
    out5 = outf.reshape(T, 8, B // 128, 8, 128)
    return jnp.transpose(out5, (2, 4, 0, 1, 3)).reshape(B, T, EMB)
